# baseline (speedup 1.0000x reference)
import jax
import jax.numpy as jnp
from jax.experimental import pallas as pl


def _gcn_conv(x, edge_index, W, b):
    n = x.shape[0]
    src = edge_index[0]
    dst = edge_index[1]
    loop = jnp.arange(n, dtype=src.dtype)
    src = jnp.concatenate([src, loop])
    dst = jnp.concatenate([dst, loop])
    h = x @ W.T
    deg = jnp.zeros((n,), dtype=h.dtype).at[dst].add(1.0)
    dis = jnp.where(deg > 0, 1.0 / jnp.sqrt(deg), 0.0)
    norm = dis[src] * dis[dst]
    msg = h[src] * norm[:, None]
    out = jnp.zeros((n, W.shape[0]), dtype=h.dtype).at[dst].add(msg)
    return out + b


def _head_kernel(h_ref, wf1_ref, bf1_ref, wf2_ref, bf2_ref, o_ref):
    h = h_ref[...]
    t = jnp.maximum(jnp.dot(h, wf1_ref[...].T, preferred_element_type=jnp.float32)
                    + bf1_ref[...][None, :], 0.0)
    o_ref[...] = (jnp.dot(t, wf2_ref[...].T, preferred_element_type=jnp.float32)
                  + bf2_ref[...][None, :])


def kernel(x, edge_index, W1, b1, W2, b2, Wf1, bf1, Wf2, bf2):
    h = _gcn_conv(x, edge_index, W1, b1)
    h = jax.nn.relu(h)
    h = _gcn_conv(h, edge_index, W2, b2)
    h = jax.nn.relu(h)
    n = h.shape[0]
    blk = 1000
    out = pl.pallas_call(
        _head_kernel,
        grid=(n // blk,),
        in_specs=[
            pl.BlockSpec((blk, h.shape[1]), lambda i: (i, 0)),
            pl.BlockSpec(Wf1.shape, lambda i: (0, 0)),
            pl.BlockSpec(bf1.shape, lambda i: (0,)),
            pl.BlockSpec(Wf2.shape, lambda i: (0, 0)),
            pl.BlockSpec(bf2.shape, lambda i: (0,)),
        ],
        out_specs=pl.BlockSpec((blk, Wf2.shape[0]), lambda i: (i, 0)),
        out_shape=jax.ShapeDtypeStruct((n, Wf2.shape[0]), jnp.float32),
    )(h, Wf1, bf1, Wf2, bf2)
    return out


# speedup vs baseline: 10.5830x; 10.5830x over previous
import functools

import jax
import jax.numpy as jnp
from jax import lax
from jax.experimental import pallas as pl
from jax.experimental.pallas import tpu as pltpu
from jax.experimental.pallas import tpu_sc as plsc

NC = 2
NS = 16
CH = 80


def _sc_mesh():
    return plsc.VectorSubcoreMesh(
        core_axis_name="c", subcore_axis_name="s", num_cores=NC, num_subcores=NS
    )


def _make_deg(n, e):
    ew = e // (NC * NS)
    nch = ew // CH
    rb = 80
    nblk = n // rb
    nrb = -(-nblk // NS)

    @functools.partial(
        pl.kernel,
        out_type=jax.ShapeDtypeStruct((NC * n, 128), jnp.float32),
        mesh=_sc_mesh(),
        scratch_types=[
            pltpu.VMEM((CH,), jnp.int32),
            pltpu.VMEM((CH, 128), jnp.float32),
            pltpu.VMEM((rb, 128), jnp.float32),
            pltpu.VMEM_SHARED((n, 128), jnp.float32),
        ],
    )
    def deg_kernel(dst_hbm, ones_hbm, zeros_hbm, out_hbm, idxv, onesv, bounce, acc):
        c = lax.axis_index("c")
        s = lax.axis_index("s")
        pltpu.sync_copy(zeros_hbm, bounce)
        pltpu.sync_copy(ones_hbm, onesv)
        for j in range(nrb):
            bid = s * nrb + j

            @pl.when(bid < nblk)
            def _():
                pltpu.sync_copy(bounce, acc.at[pl.ds(bid * rb, rb)])
        plsc.subcore_barrier()
        e0 = c * (e // 2) + s * ew

        def step(j, carry):
            pltpu.sync_copy(dst_hbm.at[pl.ds(e0 + j * CH, CH)], idxv)
            pltpu.sync_copy(onesv, acc.at[idxv], add=True)
            return carry

        lax.fori_loop(0, nch, step, 0)
        plsc.subcore_barrier()
        for j in range(nrb):
            bid = s * nrb + j

            @pl.when(bid < nblk)
            def _():
                pltpu.sync_copy(acc.at[pl.ds(bid * rb, rb)], bounce)
                pltpu.sync_copy(bounce, out_hbm.at[pl.ds(c * n + bid * rb, rb)])

    return deg_kernel


def _make_spmm_edge(n, e, f):
    ew = e // (NC * NS)
    nch = ew // CH
    rb = 80
    nblk = n // rb
    nrb = -(-nblk // NS)

    @functools.partial(
        pl.kernel,
        out_type=jax.ShapeDtypeStruct((NC * n, f), jnp.float32),
        mesh=_sc_mesh(),
        scratch_types=[
            pltpu.VMEM((CH,), jnp.int32),
            pltpu.VMEM((CH,), jnp.int32),
            pltpu.VMEM((CH, f), jnp.float32),
            pltpu.VMEM((rb, f), jnp.float32),
            pltpu.VMEM_SHARED((n, f), jnp.float32),
        ],
    )
    def spmm_kernel(src_hbm, dst_hbm, g_hbm, zeros_hbm, out_hbm,
                    srcv, dstv, rows, bounce, acc):
        c = lax.axis_index("c")
        s = lax.axis_index("s")
        pltpu.sync_copy(zeros_hbm, bounce)
        for j in range(nrb):
            bid = s * nrb + j

            @pl.when(bid < nblk)
            def _():
                pltpu.sync_copy(bounce, acc.at[pl.ds(bid * rb, rb)])
        plsc.subcore_barrier()
        e0 = c * (e // 2) + s * ew

        def step(j, carry):
            base = e0 + j * CH
            pltpu.sync_copy(src_hbm.at[pl.ds(base, CH)], srcv)
            pltpu.sync_copy(dst_hbm.at[pl.ds(base, CH)], dstv)
            pltpu.sync_copy(g_hbm.at[srcv], rows)
            pltpu.sync_copy(rows, acc.at[dstv], add=True)
            return carry

        lax.fori_loop(0, nch, step, 0)
        plsc.subcore_barrier()
        for j in range(nrb):
            bid = s * nrb + j

            @pl.when(bid < nblk)
            def _():
                pltpu.sync_copy(acc.at[pl.ds(bid * rb, rb)], bounce)
                pltpu.sync_copy(bounce, out_hbm.at[pl.ds(c * n + bid * rb, rb)])

    return spmm_kernel


def _make_spmm_feat(n, e, fh):
    ew = e // NS
    nch = ew // CH
    rb = 80
    nblk = n // rb
    nrb = -(-nblk // NS)

    @functools.partial(
        pl.kernel,
        out_type=jax.ShapeDtypeStruct((NC * n, fh), jnp.float32),
        mesh=_sc_mesh(),
        scratch_types=[
            pltpu.VMEM((CH,), jnp.int32),
            pltpu.VMEM((CH,), jnp.int32),
            pltpu.VMEM((CH,), jnp.int32),
            pltpu.VMEM((CH, fh), jnp.float32),
            pltpu.VMEM((rb, fh), jnp.float32),
            pltpu.VMEM_SHARED((n, fh), jnp.float32),
        ],
    )
    def spmm_kernel(src_hbm, dst_hbm, g_hbm, zeros_hbm, out_hbm,
                    srcv, dstv, idxg, rows, bounce, acc):
        c = lax.axis_index("c")
        s = lax.axis_index("s")
        pltpu.sync_copy(zeros_hbm, bounce)
        for j in range(nrb):
            bid = s * nrb + j

            @pl.when(bid < nblk)
            def _():
                pltpu.sync_copy(bounce, acc.at[pl.ds(bid * rb, rb)])
        plsc.subcore_barrier()
        e0 = s * ew

        def step(j, carry):
            base = e0 + j * CH
            pltpu.sync_copy(src_hbm.at[pl.ds(base, CH)], srcv)
            pltpu.sync_copy(dst_hbm.at[pl.ds(base, CH)], dstv)
            for k in range(CH // 16):
                sl = pl.ds(k * 16, 16)
                idxg[sl] = srcv[sl] * 2 + c
            pltpu.sync_copy(g_hbm.at[idxg], rows)
            pltpu.sync_copy(rows, acc.at[dstv], add=True)
            return carry

        lax.fori_loop(0, nch, step, 0)
        plsc.subcore_barrier()
        for j in range(nrb):
            bid = s * nrb + j

            @pl.when(bid < nblk)
            def _():
                pltpu.sync_copy(acc.at[pl.ds(bid * rb, rb)], bounce)
                pltpu.sync_copy(bounce, out_hbm.at[pl.ds(c * n + bid * rb, rb)])

    return spmm_kernel


def _p1_kernel(dega_ref, degb_ref, x_ref, g1_ref, d16_ref):
    deg = dega_ref[...][:, :1] + degb_ref[...][:, :1] + 1.0
    d = lax.rsqrt(deg)
    g1_ref[...] = x_ref[...] * d
    d16_ref[...] = jnp.broadcast_to(d, d16_ref.shape)


def _p2_kernel(s1a_ref, s1b_ref, g1_ref, d16_ref, w1t_ref, b1_ref, g2_ref):
    d = d16_ref[...][:, :1]
    u = (s1a_ref[...] + s1b_ref[...] + g1_ref[...]) * d
    h = (jnp.dot(u, w1t_ref[...], preferred_element_type=jnp.float32)
         + b1_ref[...])
    g2_ref[...] = jnp.maximum(h, 0.0) * d


def _p3_kernel(s2a_ref, s2b_ref, g2_ref, d16_ref, w2at_ref, w2bt_ref, b2_ref,
               wf1t_ref, bf1_ref, wf2t_ref, bf2_ref, out_ref):
    d = d16_ref[...][:, :1]
    g2 = g2_ref[...]
    u = (s2a_ref[...] + g2[:, :128]) * d
    v = (s2b_ref[...] + g2[:, 128:]) * d
    h2 = jnp.maximum(
        jnp.dot(u, w2at_ref[...], preferred_element_type=jnp.float32)
        + jnp.dot(v, w2bt_ref[...], preferred_element_type=jnp.float32)
        + b2_ref[...], 0.0)
    h3 = jnp.maximum(
        jnp.dot(h2, wf1t_ref[...], preferred_element_type=jnp.float32)
        + bf1_ref[...], 0.0)
    out_ref[...] = (jnp.dot(h3, wf2t_ref[...], preferred_element_type=jnp.float32)
                    + bf2_ref[...])


def _row_spec(blk, width):
    return pl.BlockSpec((blk, width), lambda i: (i, 0))


def _full_spec(shape):
    return pl.BlockSpec(shape, lambda i: tuple(0 for _ in shape))


def kernel(x, edge_index, W1, b1, W2, b2, Wf1, bf1, Wf2, bf2):
    n, nfeat = x.shape
    e = edge_index.shape[1]
    nhid = W1.shape[0]
    blk = 1000
    grid = (n // blk,)

    src = edge_index[0]
    dst = edge_index[1]

    deg2 = _make_deg(n, e)(
        dst,
        jnp.ones((CH, 128), jnp.float32),
        jnp.zeros((80, 128), jnp.float32),
    )

    g1, d16 = pl.pallas_call(
        _p1_kernel,
        grid=grid,
        in_specs=[_row_spec(blk, 128), _row_spec(blk, 128), _row_spec(blk, nfeat)],
        out_specs=[_row_spec(blk, nfeat), _row_spec(blk, 16)],
        out_shape=[
            jax.ShapeDtypeStruct((n, nfeat), jnp.float32),
            jax.ShapeDtypeStruct((n, 16), jnp.float32),
        ],
    )(deg2[:n], deg2[n:], x)

    zeros128a = jnp.zeros((80, nfeat), jnp.float32)
    s1 = _make_spmm_edge(n, e, nfeat)(src, dst, g1, zeros128a)

    g2 = pl.pallas_call(
        _p2_kernel,
        grid=grid,
        in_specs=[
            _row_spec(blk, nfeat), _row_spec(blk, nfeat), _row_spec(blk, nfeat),
            _row_spec(blk, 16),
            _full_spec((nfeat, nhid)), _full_spec((1, nhid)),
        ],
        out_specs=_row_spec(blk, nhid),
        out_shape=jax.ShapeDtypeStruct((n, nhid), jnp.float32),
    )(s1[:n], s1[n:], g1, d16, W1.T, b1.reshape(1, nhid))

    zeros128 = jnp.zeros((80, nhid // 2), jnp.float32)
    s2 = _make_spmm_feat(n, e, nhid // 2)(src, dst, g2.reshape(2 * n, nhid // 2),
                                          zeros128)

    out = pl.pallas_call(
        _p3_kernel,
        grid=grid,
        in_specs=[
            _row_spec(blk, 128), _row_spec(blk, 128), _row_spec(blk, nhid),
            _row_spec(blk, 16),
            _full_spec((128, nhid)), _full_spec((128, nhid)), _full_spec((1, nhid)),
            _full_spec((nhid, 128)), _full_spec((1, 128)),
            _full_spec((128, 16)), _full_spec((1, 16)),
        ],
        out_specs=_row_spec(blk, 16),
        out_shape=jax.ShapeDtypeStruct((n, 16), jnp.float32),
    )(s2[:n], s2[n:], g2, d16,
      W2[:, :128].T, W2[:, 128:].T, b2.reshape(1, nhid),
      Wf1.T, bf1.reshape(1, 128),
      Wf2.T, bf2.reshape(1, 16))
    return out
